# Initial kernel scaffold; baseline (speedup 1.0000x reference)
import jax
import jax.numpy as jnp
from jax.experimental import pallas as pl


def kernel(x, Wg, W1, W3, W2):
    raise NotImplementedError("write your pallas kernel here")


# speedup vs baseline: 2.5176x; 2.5176x over previous
import functools

import jax
import jax.numpy as jnp
from jax.experimental import pallas as pl
from jax.experimental.pallas import tpu as pltpu

T = 2048
D = 768
F = 2048
E = 8
K = 2
CAP = 256
A = T * K

_NEG_INF = float("-inf")


def _routing_kernel(logits_ref, e0_ref, e1_ref, d_ref, p0_ref, p1_ref,
                    colsum_ref, z_ref):
    logits = logits_ref[...]
    ex_iota = jax.lax.broadcasted_iota(jnp.int32, (T, E), 1)

    l0 = jnp.max(logits, axis=1, keepdims=True)
    is0 = logits == l0
    e0 = jnp.min(jnp.where(is0, ex_iota, E), axis=1, keepdims=True)
    masked = jnp.where(ex_iota == e0, _NEG_INF, logits)
    l1 = jnp.max(masked, axis=1, keepdims=True)
    is1 = masked == l1
    e1 = jnp.min(jnp.where(is1, ex_iota, E), axis=1, keepdims=True)

    u1 = jnp.exp(l1 - l0)
    den = 1.0 + u1
    p0u = 1.0 / den
    p1u = u1 / den
    s = jnp.maximum(p0u + p1u, 1e-8)
    p0_ref[...] = p0u / s
    p1_ref[...] = p1u / s

    e0_ref[...] = e0
    e1_ref[...] = e1
    d_ref[...] = l0 - l1

    exps = jnp.exp(logits - l0)
    den8 = jnp.sum(exps, axis=1, keepdims=True)
    colsum_ref[...] = jnp.sum(exps / den8, axis=0, keepdims=True)

    lse = l0 + jnp.log(den8)
    z_ref[...] = jnp.sum(lse * lse, axis=0, keepdims=True) * (0.001 / T)


_BI = 128


def _rank_kernel(scol_ref, ecol_ref, srow_ref, erow_ref, rank_ref, keep_ref):
    i = pl.program_id(0)
    scol = scol_ref[...]
    ecol = ecol_ref[...]
    srow = srow_ref[...]
    erow = erow_ref[...]
    icol = i * _BI + jax.lax.broadcasted_iota(jnp.int32, (_BI, 1), 0)
    irow = jax.lax.broadcasted_iota(jnp.int32, (1, A), 1)

    same_e = erow == ecol
    beats = (srow > scol) | ((srow == scol) & (irow < icol))
    cnt = jnp.sum(jnp.where(same_e & beats, 1.0, 0.0), axis=1, keepdims=True)
    rank = cnt.astype(jnp.int32)
    rank_ref[...] = rank
    keep_ref[...] = (rank < CAP).astype(jnp.int32)


_FB = 2
_FBLK = F // _FB


def _expert_kernel(x_ref, w1_ref, w3_ref, w2_ref,
                   e0r_ref, e1r_ref, r0r_ref, r1r_ref, k0r_ref, k1r_ref,
                   e0c_ref, e1c_ref, r0c_ref, r1c_ref, k0c_ref, k1c_ref,
                   p0c_ref, p1c_ref, colsum_ref,
                   out_ref, lb_ref, buf_ref, y_ref):
    e = pl.program_id(0)
    fb = pl.program_id(1)

    @pl.when(fb == 0)
    def _dispatch():
        q_row = jnp.where(
            (e0r_ref[...] == e) & (k0r_ref[...] == 1), r0r_ref[...],
            jnp.where((e1r_ref[...] == e) & (k1r_ref[...] == 1),
                      r1r_ref[...], -1))
        slot_iota = jax.lax.broadcasted_iota(jnp.int32, (CAP, T), 0)
        disp = (slot_iota == q_row).astype(jnp.float32)
        buf_ref[...] = jax.lax.dot_general(
            disp, x_ref[...], (((1,), (0,)), ((), ())),
            precision=jax.lax.Precision.HIGHEST)

    buf = buf_ref[...]
    w1 = w1_ref[0]
    w3 = w3_ref[0]
    w2 = w2_ref[0]
    h1 = jax.lax.dot_general(buf, w1, (((1,), (1,)), ((), ())))
    h3 = jax.lax.dot_general(buf, w3, (((1,), (1,)), ((), ())))
    h = h1 * jax.lax.logistic(h1) * h3
    y_part = jax.lax.dot_general(h, w2, (((1,), (1,)), ((), ())))

    @pl.when(fb == 0)
    def _y_init():
        y_ref[...] = y_part

    @pl.when(fb != 0)
    def _y_acc():
        y_ref[...] += y_part

    @pl.when((e == 0) & (fb == 0))
    def _init():
        out_ref[...] = jnp.zeros((T, D), jnp.float32)
        lb_ref[...] = jnp.zeros((1, 1), jnp.float32)

    @pl.when(fb == _FB - 1)
    def _combine():
        q_col = jnp.where(
            (e0c_ref[...] == e) & (k0c_ref[...] == 1), r0c_ref[...],
            jnp.where((e1c_ref[...] == e) & (k1c_ref[...] == 1),
                      r1c_ref[...], -1))
        w_col = jnp.where(
            (e0c_ref[...] == e) & (k0c_ref[...] == 1), p0c_ref[...],
            jnp.where((e1c_ref[...] == e) & (k1c_ref[...] == 1),
                      p1c_ref[...], 0.0))
        tok_iota = jax.lax.broadcasted_iota(jnp.int32, (T, CAP), 1)
        comb = (tok_iota == q_col).astype(jnp.float32)
        contrib = jax.lax.dot_general(
            comb, y_ref[...], (((1,), (0,)), ((), ())),
            precision=jax.lax.Precision.HIGHEST)
        out_ref[...] += w_col * contrib

        cnt = jnp.sum(jnp.where((e0c_ref[...] == e) & (k0c_ref[...] == 1),
                                1.0, 0.0), axis=0, keepdims=True)
        lane_iota = jax.lax.broadcasted_iota(jnp.int32, (1, E), 1)
        cs_e = jnp.sum(jnp.where(lane_iota == e, colsum_ref[...], 0.0),
                       axis=1, keepdims=True)
        lb_ref[...] += cs_e * cnt * (0.01 / (T * E))


@jax.jit
def kernel(x, Wg, W1, W3, W2):
    logits = x @ Wg.T

    e0, e1, d, p0, p1, colsum, z = pl.pallas_call(
        _routing_kernel,
        out_shape=(
            jax.ShapeDtypeStruct((T, 1), jnp.int32),
            jax.ShapeDtypeStruct((T, 1), jnp.int32),
            jax.ShapeDtypeStruct((T, 1), jnp.float32),
            jax.ShapeDtypeStruct((T, 1), jnp.float32),
            jax.ShapeDtypeStruct((T, 1), jnp.float32),
            jax.ShapeDtypeStruct((1, E), jnp.float32),
            jax.ShapeDtypeStruct((1, 1), jnp.float32),
        ),
    )(logits)

    s_flat = jnp.concatenate([d, -d], axis=1).reshape(A, 1)
    e_flat = jnp.concatenate([e0, e1], axis=1).reshape(A, 1)

    rank, keep = pl.pallas_call(
        _rank_kernel,
        grid=(A // _BI,),
        in_specs=[
            pl.BlockSpec((_BI, 1), lambda i: (i, 0)),
            pl.BlockSpec((_BI, 1), lambda i: (i, 0)),
            pl.BlockSpec((1, A), lambda i: (0, 0)),
            pl.BlockSpec((1, A), lambda i: (0, 0)),
        ],
        out_shape=(
            jax.ShapeDtypeStruct((A, 1), jnp.int32),
            jax.ShapeDtypeStruct((A, 1), jnp.int32),
        ),
        out_specs=(
            pl.BlockSpec((_BI, 1), lambda i: (i, 0)),
            pl.BlockSpec((_BI, 1), lambda i: (i, 0)),
        ),
    )(s_flat, e_flat, s_flat.reshape(1, A), e_flat.reshape(1, A))

    rank2 = rank.reshape(T, K)
    keep2 = keep.reshape(T, K)
    r0c, r1c = rank2[:, 0:1], rank2[:, 1:2]
    k0c, k1c = keep2[:, 0:1], keep2[:, 1:2]

    row = lambda a: a.reshape(1, T)
    out, lb = pl.pallas_call(
        _expert_kernel,
        grid=(E, _FB),
        in_specs=[
            pl.BlockSpec((T, D), lambda e, fb: (0, 0)),
            pl.BlockSpec((1, _FBLK, D), lambda e, fb: (e, fb, 0)),
            pl.BlockSpec((1, _FBLK, D), lambda e, fb: (e, fb, 0)),
            pl.BlockSpec((1, D, _FBLK), lambda e, fb: (e, 0, fb)),
        ] + [pl.BlockSpec((1, T), lambda e, fb: (0, 0))] * 6
          + [pl.BlockSpec((T, 1), lambda e, fb: (0, 0))] * 8
          + [pl.BlockSpec((1, E), lambda e, fb: (0, 0))],
        out_shape=(
            jax.ShapeDtypeStruct((T, D), jnp.float32),
            jax.ShapeDtypeStruct((1, 1), jnp.float32),
        ),
        out_specs=(
            pl.BlockSpec((T, D), lambda e, fb: (0, 0)),
            pl.BlockSpec((1, 1), lambda e, fb: (0, 0)),
        ),
        scratch_shapes=[
            pltpu.VMEM((CAP, D), jnp.float32),
            pltpu.VMEM((CAP, D), jnp.float32),
        ],
    )(x, W1, W3, W2,
      row(e0), row(e1), row(r0c), row(r1c), row(k0c), row(k1c),
      e0, e1, r0c, r1c, k0c, k1c, p0, p1, colsum)

    return out, lb.reshape(()), z.reshape(())


# speedup vs baseline: 3.6158x; 1.4362x over previous
import functools

import jax
import jax.numpy as jnp
from jax.experimental import pallas as pl
from jax.experimental.pallas import tpu as pltpu

T = 2048
D = 768
F = 2048
E = 8
K = 2
CAP = 256
A = T * K

_NEG_INF = float("-inf")


def _routing_kernel(logits_ref, e0_ref, e1_ref, d_ref, p0_ref, p1_ref,
                    colsum_ref, z_ref):
    logits = logits_ref[...]
    ex_iota = jax.lax.broadcasted_iota(jnp.int32, (T, E), 1)

    l0 = jnp.max(logits, axis=1, keepdims=True)
    is0 = logits == l0
    e0 = jnp.min(jnp.where(is0, ex_iota, E), axis=1, keepdims=True)
    masked = jnp.where(ex_iota == e0, _NEG_INF, logits)
    l1 = jnp.max(masked, axis=1, keepdims=True)
    is1 = masked == l1
    e1 = jnp.min(jnp.where(is1, ex_iota, E), axis=1, keepdims=True)

    u1 = jnp.exp(l1 - l0)
    den = 1.0 + u1
    p0u = 1.0 / den
    p1u = u1 / den
    s = jnp.maximum(p0u + p1u, 1e-8)
    p0_ref[...] = p0u / s
    p1_ref[...] = p1u / s

    e0_ref[...] = e0
    e1_ref[...] = e1
    d_ref[...] = l0 - l1

    exps = jnp.exp(logits - l0)
    den8 = jnp.sum(exps, axis=1, keepdims=True)
    colsum_ref[...] = jnp.sum(exps / den8, axis=0, keepdims=True)

    lse = l0 + jnp.log(den8)
    z_ref[...] = jnp.sum(lse * lse, axis=0, keepdims=True) * (0.001 / T)


_BI = 128


def _rank_kernel(scol_ref, ecol_ref, srow_ref, erow_ref, rank_ref, keep_ref):
    i = pl.program_id(0)
    scol = scol_ref[...]
    ecol = ecol_ref[...]
    srow = srow_ref[...]
    erow = erow_ref[...]
    icol = i * _BI + jax.lax.broadcasted_iota(jnp.int32, (_BI, 1), 0)
    irow = jax.lax.broadcasted_iota(jnp.int32, (1, A), 1)

    same_e = erow == ecol
    beats = (srow > scol) | ((srow == scol) & (irow < icol))
    cnt = jnp.sum(jnp.where(same_e & beats, 1.0, 0.0), axis=1, keepdims=True)
    rank = cnt.astype(jnp.int32)
    rank_ref[...] = rank
    keep_ref[...] = (rank < CAP).astype(jnp.int32)


_FB = 2
_FBLK = F // _FB


def _expert_kernel(x_ref, w1_ref, w3_ref, w2_ref,
                   e0r_ref, e1r_ref, r0r_ref, r1r_ref, k0r_ref, k1r_ref,
                   e0c_ref, e1c_ref, r0c_ref, r1c_ref, k0c_ref, k1c_ref,
                   p0c_ref, p1c_ref, colsum_ref,
                   out_ref, lb_ref, buf_ref, y_ref):
    e = pl.program_id(0)
    fb = pl.program_id(1)

    @pl.when(fb == 0)
    def _dispatch():
        q_row = jnp.where(
            (e0r_ref[...] == e) & (k0r_ref[...] == 1), r0r_ref[...],
            jnp.where((e1r_ref[...] == e) & (k1r_ref[...] == 1),
                      r1r_ref[...], -1))
        slot_iota = jax.lax.broadcasted_iota(jnp.int32, (CAP, T), 0)
        disp = (slot_iota == q_row).astype(jnp.bfloat16)
        buf_ref[...] = jax.lax.dot_general(
            disp, x_ref[...].astype(jnp.bfloat16), (((1,), (0,)), ((), ())),
            preferred_element_type=jnp.float32)

    buf = buf_ref[...].astype(jnp.bfloat16)
    w1 = w1_ref[0].astype(jnp.bfloat16)
    w3 = w3_ref[0].astype(jnp.bfloat16)
    w2 = w2_ref[0].astype(jnp.bfloat16)
    h1 = jax.lax.dot_general(buf, w1, (((1,), (1,)), ((), ())),
                             preferred_element_type=jnp.float32)
    h3 = jax.lax.dot_general(buf, w3, (((1,), (1,)), ((), ())),
                             preferred_element_type=jnp.float32)
    h = h1 * jax.lax.logistic(h1) * h3
    y_part = jax.lax.dot_general(h.astype(jnp.bfloat16), w2,
                                 (((1,), (1,)), ((), ())),
                                 preferred_element_type=jnp.float32)

    @pl.when(fb == 0)
    def _y_init():
        y_ref[...] = y_part

    @pl.when(fb != 0)
    def _y_acc():
        y_ref[...] += y_part

    @pl.when((e == 0) & (fb == 0))
    def _init():
        out_ref[...] = jnp.zeros((T, D), jnp.float32)
        lb_ref[...] = jnp.zeros((1, 1), jnp.float32)

    @pl.when(fb == _FB - 1)
    def _combine():
        q_col = jnp.where(
            (e0c_ref[...] == e) & (k0c_ref[...] == 1), r0c_ref[...],
            jnp.where((e1c_ref[...] == e) & (k1c_ref[...] == 1),
                      r1c_ref[...], -1))
        w_col = jnp.where(
            (e0c_ref[...] == e) & (k0c_ref[...] == 1), p0c_ref[...],
            jnp.where((e1c_ref[...] == e) & (k1c_ref[...] == 1),
                      p1c_ref[...], 0.0))
        tok_iota = jax.lax.broadcasted_iota(jnp.int32, (T, CAP), 1)
        comb = (tok_iota == q_col).astype(jnp.bfloat16)
        contrib = jax.lax.dot_general(
            comb, y_ref[...].astype(jnp.bfloat16), (((1,), (0,)), ((), ())),
            preferred_element_type=jnp.float32)
        out_ref[...] += w_col * contrib

        cnt = jnp.sum(jnp.where((e0c_ref[...] == e) & (k0c_ref[...] == 1),
                                1.0, 0.0), axis=0, keepdims=True)
        lane_iota = jax.lax.broadcasted_iota(jnp.int32, (1, E), 1)
        cs_e = jnp.sum(jnp.where(lane_iota == e, colsum_ref[...], 0.0),
                       axis=1, keepdims=True)
        lb_ref[...] += cs_e * cnt * (0.01 / (T * E))


@jax.jit
def kernel(x, Wg, W1, W3, W2):
    logits = x @ Wg.T

    e0, e1, d, p0, p1, colsum, z = pl.pallas_call(
        _routing_kernel,
        out_shape=(
            jax.ShapeDtypeStruct((T, 1), jnp.int32),
            jax.ShapeDtypeStruct((T, 1), jnp.int32),
            jax.ShapeDtypeStruct((T, 1), jnp.float32),
            jax.ShapeDtypeStruct((T, 1), jnp.float32),
            jax.ShapeDtypeStruct((T, 1), jnp.float32),
            jax.ShapeDtypeStruct((1, E), jnp.float32),
            jax.ShapeDtypeStruct((1, 1), jnp.float32),
        ),
    )(logits)

    s_flat = jnp.concatenate([d, -d], axis=1).reshape(A, 1)
    e_flat = jnp.concatenate([e0, e1], axis=1).reshape(A, 1)

    rank, keep = pl.pallas_call(
        _rank_kernel,
        grid=(A // _BI,),
        in_specs=[
            pl.BlockSpec((_BI, 1), lambda i: (i, 0)),
            pl.BlockSpec((_BI, 1), lambda i: (i, 0)),
            pl.BlockSpec((1, A), lambda i: (0, 0)),
            pl.BlockSpec((1, A), lambda i: (0, 0)),
        ],
        out_shape=(
            jax.ShapeDtypeStruct((A, 1), jnp.int32),
            jax.ShapeDtypeStruct((A, 1), jnp.int32),
        ),
        out_specs=(
            pl.BlockSpec((_BI, 1), lambda i: (i, 0)),
            pl.BlockSpec((_BI, 1), lambda i: (i, 0)),
        ),
    )(s_flat, e_flat, s_flat.reshape(1, A), e_flat.reshape(1, A))

    rank2 = rank.reshape(T, K)
    keep2 = keep.reshape(T, K)
    r0c, r1c = rank2[:, 0:1], rank2[:, 1:2]
    k0c, k1c = keep2[:, 0:1], keep2[:, 1:2]

    row = lambda a: a.reshape(1, T)
    out, lb = pl.pallas_call(
        _expert_kernel,
        grid=(E, _FB),
        in_specs=[
            pl.BlockSpec((T, D), lambda e, fb: (0, 0)),
            pl.BlockSpec((1, _FBLK, D), lambda e, fb: (e, fb, 0)),
            pl.BlockSpec((1, _FBLK, D), lambda e, fb: (e, fb, 0)),
            pl.BlockSpec((1, D, _FBLK), lambda e, fb: (e, 0, fb)),
        ] + [pl.BlockSpec((1, T), lambda e, fb: (0, 0))] * 6
          + [pl.BlockSpec((T, 1), lambda e, fb: (0, 0))] * 8
          + [pl.BlockSpec((1, E), lambda e, fb: (0, 0))],
        out_shape=(
            jax.ShapeDtypeStruct((T, D), jnp.float32),
            jax.ShapeDtypeStruct((1, 1), jnp.float32),
        ),
        out_specs=(
            pl.BlockSpec((T, D), lambda e, fb: (0, 0)),
            pl.BlockSpec((1, 1), lambda e, fb: (0, 0)),
        ),
        scratch_shapes=[
            pltpu.VMEM((CAP, D), jnp.float32),
            pltpu.VMEM((CAP, D), jnp.float32),
        ],
    )(x, W1, W3, W2,
      row(e0), row(e1), row(r0c), row(r1c), row(k0c), row(k1c),
      e0, e1, r0c, r1c, k0c, k1c, p0, p1, colsum)

    return out, lb.reshape(()), z.reshape(())
